# baseline (speedup 1.0000x reference)
import functools

import jax
import jax.numpy as jnp
from jax import lax
from jax.experimental import pallas as pl
from jax.experimental.pallas import tpu as pltpu
from jax.experimental.pallas import tpu_sc as plsc

_NC = 2
_NS = 16
_L = 16


def _edge_lin_body(ea_ref, w_ref, b_ref, o_ref):
  a = ea_ref[...]
  o_ref[...] = (
      jnp.dot(a, w_ref[...], preferred_element_type=jnp.float32) + b_ref[...]
  )


def _edge_lin(edge_attr, w, b, blk=2560):
  e, h = edge_attr.shape
  d1 = w.shape[1]
  return pl.pallas_call(
      _edge_lin_body,
      grid=(e // blk,),
      in_specs=[
          pl.BlockSpec((blk, h), lambda i: (i, 0)),
          pl.BlockSpec((h, d1), lambda i: (0, 0)),
          pl.BlockSpec((1, d1), lambda i: (0, 0)),
      ],
      out_specs=pl.BlockSpec((blk, d1), lambda i: (i, 0)),
      out_shape=jax.ShapeDtypeStruct((e, d1), jnp.float32),
  )(edge_attr, w, b.reshape(1, d1))


def _mlp_body(d, final_relu, pad_to, agg_ref, x_ref, w1_ref, b1_ref, g_ref,
              be_ref, w2_ref, b2_ref, o_ref):
  x = x_ref[...]
  if x.shape[1] > d:
    x = x[:, :d]
  out = agg_ref[0] + agg_ref[1] + x
  h = jnp.dot(out, w1_ref[...], preferred_element_type=jnp.float32) + b1_ref[...]
  mu = jnp.mean(h, axis=0, keepdims=True)
  var = jnp.mean((h - mu) * (h - mu), axis=0, keepdims=True)
  h = (h - mu) / jnp.sqrt(var + 1e-5) * g_ref[...] + be_ref[...]
  h = jnp.maximum(h, 0.0)
  o = jnp.dot(h, w2_ref[...], preferred_element_type=jnp.float32) + b2_ref[...]
  if final_relu:
    o = jnp.maximum(o, 0.0)
  if pad_to > o.shape[1]:
    o = jnp.concatenate(
        [o, jnp.zeros((o.shape[0], pad_to - o.shape[1]), jnp.float32)], axis=1)
  o_ref[...] = o


def _mlp(agg, x, w1, b1, g, be, w2, b2, final_relu, pad_to=0):
  n = x.shape[0]
  dx = x.shape[1]
  d = w1.shape[0]
  dh = w1.shape[1]
  do = w2.shape[1]
  dout = max(do, pad_to)
  return pl.pallas_call(
      functools.partial(_mlp_body, d, final_relu, pad_to),
      grid=(1,),
      in_specs=[
          pl.BlockSpec((2, n, d), lambda i: (0, 0, 0)),
          pl.BlockSpec((n, dx), lambda i: (0, 0)),
          pl.BlockSpec((d, dh), lambda i: (0, 0)),
          pl.BlockSpec((1, dh), lambda i: (0, 0)),
          pl.BlockSpec((1, dh), lambda i: (0, 0)),
          pl.BlockSpec((1, dh), lambda i: (0, 0)),
          pl.BlockSpec((dh, do), lambda i: (0, 0)),
          pl.BlockSpec((1, do), lambda i: (0, 0)),
      ],
      out_specs=pl.BlockSpec((n, dout), lambda i: (0, 0)),
      out_shape=jax.ShapeDtypeStruct((n, dout), jnp.float32),
  )(agg, x, w1, b1.reshape(1, dh), g.reshape(1, dh), be.reshape(1, dh),
    w2, b2.reshape(1, do))


def _make_sc_conv(n, e, d, dt):
  nw = _NC * _NS
  ew = e // nw
  k = 80
  nch = ew // k
  npad = -(-n // (k * _NS)) * (k * _NS)
  rt = npad // _NS
  grp = d // _L
  assert ew * nw == e and nch * k == ew and rt % k == 0 and dt >= d
  assert nch % 2 == 1 and nch >= 3

  mesh = plsc.VectorSubcoreMesh(core_axis_name="c", subcore_axis_name="s",
                                num_cores=_NC, num_subcores=_NS)

  def body(x_hbm, src_hbm, dst_hbm, ea_hbm, out_hbm,
           acc, srcb0, srcb1, dstb0, dstb1, xjb0, xjb1, msgb0, msgb1,
           ps0, ps1, gs0, gs1, es0, es1, ds0, ds1, ss0, ss1):
    c = lax.axis_index("c")
    s = lax.axis_index("s")
    srcb = (srcb0, srcb1)
    dstb = (dstb0, dstb1)
    xjb = (xjb0, xjb1)
    msgb = (msgb0, msgb1)
    ps = (ps0, ps1)
    gs = (gs0, gs1)
    es = (es0, es1)
    ds = (ds0, ds1)
    ss = (ss0, ss1)
    base = (c * _NS + s) * ew

    def issue_src(j, b):
      pltpu.async_copy(src_hbm.at[pl.ds(base + j * k, k)], srcb[b], ps[b])

    def wait_src(b):
      pltpu.make_async_copy(src_hbm.at[pl.ds(base, k)], srcb[b], ps[b]).wait()

    def issue_in(j, b):
      pltpu.async_copy(x_hbm.at[srcb[b]], xjb[b], gs[b])
      pltpu.async_copy(ea_hbm.at[pl.ds(base + j * k, k)], msgb[b], es[b])
      pltpu.async_copy(dst_hbm.at[pl.ds(base + j * k, k)], dstb[b], ds[b])

    def wait_in(b):
      pltpu.make_async_copy(x_hbm.at[srcb[b]], xjb[b], gs[b]).wait()
      pltpu.make_async_copy(ea_hbm.at[pl.ds(base, k)], msgb[b], es[b]).wait()

    def wait_dst(b):
      pltpu.make_async_copy(dst_hbm.at[pl.ds(base, k)], dstb[b], ds[b]).wait()

    def compute(b):
      @plsc.parallel_loop(0, k)
      def _(r):
        for v in range(grp):
          sl = pl.ds(v * _L, _L)
          msgb[b][r, sl] = jnp.maximum(msgb[b][r, sl] + xjb[b][r, sl], 0.0)

    def issue_scatter(b):
      pltpu.async_copy(msgb[b], acc.at[dstb[b]], ss[b], add=True)

    def wait_scatter(b):
      pltpu.make_async_copy(msgb[b], acc.at[dstb[b]], ss[b]).wait()

    pltpu.sync_copy(src_hbm.at[pl.ds(base, k)], srcb0)
    pltpu.async_copy(ea_hbm.at[pl.ds(base, k)], msgb0, es0)
    pltpu.async_copy(dst_hbm.at[pl.ds(base, k)], dstb0, ds0)
    issue_src(1, 1)

    def zrow(r, carry):
      for v in range(dt // _L):
        xjb0[r, pl.ds(v * _L, _L)] = jnp.zeros((_L,), jnp.float32)
      return carry
    lax.fori_loop(0, k, zrow, 0)
    for t in range(rt // k):
      pltpu.sync_copy(xjb0.at[..., pl.ds(0, d)] if dt > d else xjb0,
                      acc.at[pl.ds(s * rt + t * k, k)])
    plsc.subcore_barrier()

    pltpu.async_copy(x_hbm.at[srcb0], xjb0, gs0)
    wait_src(1)
    issue_in(1, 1)
    wait_in(0)
    issue_src(2, 0)
    compute(0)
    wait_dst(0)
    issue_scatter(0)

    def pair(p, carry):
      for t in range(2):
        j = 1 + 2 * p + t
        b = (1 + t) % 2
        nb = 1 - b
        wait_scatter(nb)

        @pl.when(j < nch - 1)
        def _():
          wait_src(nb)
          issue_in(j + 1, nb)
        wait_in(b)

        @pl.when(j < nch - 2)
        def _():
          issue_src(j + 2, b)
        compute(b)
        wait_dst(b)
        issue_scatter(b)
      return carry
    lax.fori_loop(0, (nch - 1) // 2, pair, 0)
    wait_scatter((nch - 1) % 2)

    plsc.subcore_barrier()
    pltpu.sync_copy(acc.at[pl.ds(s * rt, rt)],
                    out_hbm.at[c, pl.ds(s * rt, rt)])

  return pl.kernel(
      body,
      out_type=jax.ShapeDtypeStruct((_NC, npad, d), jnp.float32),
      mesh=mesh,
      scratch_types=[
          pltpu.VMEM_SHARED((npad, d), jnp.float32),
          pltpu.VMEM((k,), jnp.int32),
          pltpu.VMEM((k,), jnp.int32),
          pltpu.VMEM((k,), jnp.int32),
          pltpu.VMEM((k,), jnp.int32),
          pltpu.VMEM((k, dt), jnp.float32),
          pltpu.VMEM((k, dt), jnp.float32),
          pltpu.VMEM((k, d), jnp.float32),
          pltpu.VMEM((k, d), jnp.float32),
          pltpu.SemaphoreType.DMA,
          pltpu.SemaphoreType.DMA,
          pltpu.SemaphoreType.DMA,
          pltpu.SemaphoreType.DMA,
          pltpu.SemaphoreType.DMA,
          pltpu.SemaphoreType.DMA,
          pltpu.SemaphoreType.DMA,
          pltpu.SemaphoreType.DMA,
          pltpu.SemaphoreType.DMA,
          pltpu.SemaphoreType.DMA,
      ],
  )


def kernel(x, edge_index, edge_attr,
           lin1_W, lin1_b, m1_W1, m1_b1, m1_g, m1_be, m1_W2, m1_b2,
           lin2_W, lin2_b, m2_W1, m2_b1, m2_g, m2_be, m2_W2, m2_b2):
  n, d_in = x.shape
  e, h_dim = edge_attr.shape
  src = edge_index[0]
  dst = edge_index[1]

  pad = d_in - h_dim
  lin2_Wp = jnp.concatenate([lin2_W, jnp.zeros((h_dim, pad), jnp.float32)], 1)
  lin2_bp = jnp.concatenate([lin2_b, jnp.zeros((pad,), jnp.float32)], 0)
  m2_W1p = jnp.concatenate([m2_W1, jnp.zeros((pad, m2_W1.shape[1]),
                                             jnp.float32)], 0)

  ea1 = _edge_lin(edge_attr, lin1_W, lin1_b)
  ea2 = _edge_lin(edge_attr, lin2_Wp, lin2_bp)
  agg1 = _make_sc_conv(n, e, d_in, d_in)(x, src, dst, ea1)
  h = _mlp(agg1, x, m1_W1, m1_b1, m1_g, m1_be, m1_W2, m1_b2,
           final_relu=True, pad_to=d_in)
  agg2 = _make_sc_conv(n, e, d_in, d_in)(h, src, dst, ea2)
  return _mlp(agg2, h, m2_W1p, m2_b1, m2_g, m2_be, m2_W2, m2_b2,
              final_relu=False)


# speedup vs baseline: 1.0644x; 1.0644x over previous
import functools

import jax
import jax.numpy as jnp
from jax import lax
from jax.experimental import pallas as pl
from jax.experimental.pallas import tpu as pltpu
from jax.experimental.pallas import tpu_sc as plsc

_NC = 2
_NS = 16
_L = 16


def _edge_lin_body(blk, grp, ea_ref, w1_ref, b1_ref, w2_ref, b2_ref,
                   o1_ref, o2_ref):
  a = ea_ref[...]
  o1_ref[...] = (
      jnp.dot(a, w1_ref[...], preferred_element_type=jnp.float32) + b1_ref[...]
  )
  o2 = jnp.dot(a, w2_ref[...], preferred_element_type=jnp.float32) + b2_ref[...]
  h2 = o2.shape[1]
  o3 = o2.reshape(blk // grp, grp, h2)
  packed = jnp.concatenate([o3[:, :grp // 2, :], o3[:, grp // 2:, :]], axis=2)
  o2_ref[...] = packed.reshape(blk // 2, 2 * h2)


def _edge_lin(edge_attr, w1, b1, w2, b2, grp, blk=2560):
  e, h = edge_attr.shape
  d1 = w1.shape[1]
  d2 = w2.shape[1]
  assert blk % grp == 0 and grp % 2 == 0
  return pl.pallas_call(
      functools.partial(_edge_lin_body, blk, grp),
      grid=(e // blk,),
      in_specs=[
          pl.BlockSpec((blk, h), lambda i: (i, 0)),
          pl.BlockSpec((h, d1), lambda i: (0, 0)),
          pl.BlockSpec((1, d1), lambda i: (0, 0)),
          pl.BlockSpec((h, d2), lambda i: (0, 0)),
          pl.BlockSpec((1, d2), lambda i: (0, 0)),
      ],
      out_specs=[
          pl.BlockSpec((blk, d1), lambda i: (i, 0)),
          pl.BlockSpec((blk // 2, 2 * d2), lambda i: (i, 0)),
      ],
      out_shape=[
          jax.ShapeDtypeStruct((e, d1), jnp.float32),
          jax.ShapeDtypeStruct((e // 2, 2 * d2), jnp.float32),
      ],
  )(edge_attr, w1, b1.reshape(1, d1), w2, b2.reshape(1, d2))


def _mlp_body(d, final_relu, pad_to, agg_ref, x_ref, w1_ref, b1_ref, g_ref,
              be_ref, w2_ref, b2_ref, o_ref):
  x = x_ref[...]
  if x.shape[1] > d:
    x = x[:, :d]
  out = agg_ref[0] + agg_ref[1] + x
  h = jnp.dot(out, w1_ref[...], preferred_element_type=jnp.float32) + b1_ref[...]
  mu = jnp.mean(h, axis=0, keepdims=True)
  var = jnp.mean((h - mu) * (h - mu), axis=0, keepdims=True)
  h = (h - mu) / jnp.sqrt(var + 1e-5) * g_ref[...] + be_ref[...]
  h = jnp.maximum(h, 0.0)
  o = jnp.dot(h, w2_ref[...], preferred_element_type=jnp.float32) + b2_ref[...]
  if final_relu:
    o = jnp.maximum(o, 0.0)
  if pad_to > o.shape[1]:
    o = jnp.concatenate(
        [o, jnp.zeros((o.shape[0], pad_to - o.shape[1]), jnp.float32)], axis=1)
  o_ref[...] = o


def _mlp(agg, x, w1, b1, g, be, w2, b2, final_relu, pad_to=0):
  n = x.shape[0]
  dx = x.shape[1]
  d = w1.shape[0]
  dh = w1.shape[1]
  do = w2.shape[1]
  dout = max(do, pad_to)
  return pl.pallas_call(
      functools.partial(_mlp_body, d, final_relu, pad_to),
      grid=(1,),
      in_specs=[
          pl.BlockSpec((2, n, d), lambda i: (0, 0, 0)),
          pl.BlockSpec((n, dx), lambda i: (0, 0)),
          pl.BlockSpec((d, dh), lambda i: (0, 0)),
          pl.BlockSpec((1, dh), lambda i: (0, 0)),
          pl.BlockSpec((1, dh), lambda i: (0, 0)),
          pl.BlockSpec((1, dh), lambda i: (0, 0)),
          pl.BlockSpec((dh, do), lambda i: (0, 0)),
          pl.BlockSpec((1, do), lambda i: (0, 0)),
      ],
      out_specs=pl.BlockSpec((n, dout), lambda i: (0, 0)),
      out_shape=jax.ShapeDtypeStruct((n, dout), jnp.float32),
  )(agg, x, w1, b1.reshape(1, dh), g.reshape(1, dh), be.reshape(1, dh),
    w2, b2.reshape(1, do))


def _make_sc_conv(n, e, d, k, packed_ea):
  nw = _NC * _NS
  ew = e // nw
  nch = ew // k
  npad = -(-n // (k * _NS)) * (k * _NS)
  rt = npad // _NS
  grp = d // _L
  dm = d // 2
  kk = k // 2
  assert ew * nw == e and nch * k == ew and rt % k == 0
  assert nch >= 3 and k % 8 == 0 and k <= 128

  mesh = plsc.VectorSubcoreMesh(core_axis_name="c", subcore_axis_name="s",
                                num_cores=_NC, num_subcores=_NS)
  ea_rows = kk if packed_ea else k

  def body(x_hbm, src_hbm, dst_hbm, ea_hbm, out_hbm,
           acc, srcb0, srcb1, dstb0, dstb1, xjb0, xjb1, msgb0, msgb1,
           eab0, ps0, ps1, gs0, gs1, es0, es1, ds0, ds1, ss0, ss1):
    c = lax.axis_index("c")
    s = lax.axis_index("s")
    srcb = (srcb0, srcb1)
    dstb = (dstb0, dstb1)
    xjb = (xjb0, xjb1)
    msgb = (msgb0, msgb1)
    eab = (eab0, eab0) if packed_ea else (msgb0, msgb1)
    es = (es0, es0) if packed_ea else (es0, es1)
    ps = (ps0, ps1)
    gs = (gs0, gs1)
    ds = (ds0, ds1)
    ss = (ss0, ss1)
    base = (c * _NS + s) * ew
    base_ea = base // 2 if packed_ea else base

    def issue_src(j, b):
      pltpu.async_copy(src_hbm.at[pl.ds(base + j * k, k)], srcb[b], ps[b])

    def wait_src(b):
      pltpu.make_async_copy(src_hbm.at[pl.ds(base, k)], srcb[b], ps[b]).wait()

    def issue_ea(j, b):
      off = pl.multiple_of(base_ea + j * ea_rows, 8)
      pltpu.async_copy(ea_hbm.at[pl.ds(off, ea_rows)], eab[b], es[b])

    def wait_ea(b):
      off = pl.multiple_of(base_ea, 8)
      pltpu.make_async_copy(ea_hbm.at[pl.ds(off, ea_rows)], eab[b],
                            es[b]).wait()

    def issue_dst(j, b):
      pltpu.async_copy(dst_hbm.at[pl.ds(base + j * k, k)], dstb[b], ds[b])

    def issue_gather_dst(j, b):
      pltpu.async_copy(x_hbm.at[srcb[b]], xjb[b], gs[b])
      issue_dst(j, b)

    def wait_gather(b):
      pltpu.make_async_copy(x_hbm.at[srcb[b]], xjb[b], gs[b]).wait()

    def wait_dst(b):
      pltpu.make_async_copy(dst_hbm.at[pl.ds(base, k)], dstb[b], ds[b]).wait()

    def compute(b):
      if packed_ea:
        @plsc.parallel_loop(0, kk)
        def _(r):
          for half in range(2):
            for v in range(dm // _L):
              so = pl.ds(v * _L, _L)
              eo = pl.ds(half * dm + v * _L, _L)
              msgb[b][half * kk + r, so] = jnp.maximum(
                  xjb[b][half * kk + r, so] + eab0[r, eo], 0.0)
      else:
        @plsc.parallel_loop(0, k)
        def _(r):
          for v in range(grp):
            sl = pl.ds(v * _L, _L)
            msgb[b][r, sl] = jnp.maximum(msgb[b][r, sl] + xjb[b][r, sl], 0.0)

    def issue_scatter(b):
      pltpu.async_copy(msgb[b], acc.at[dstb[b]], ss[b], add=True)

    def wait_scatter(b):
      pltpu.make_async_copy(msgb[b], acc.at[dstb[b]], ss[b]).wait()

    pltpu.sync_copy(src_hbm.at[pl.ds(base, k)], srcb0)
    issue_ea(0, 0)
    issue_dst(0, 0)
    issue_src(1, 1)

    def zrow(r, carry):
      for v in range(grp):
        xjb0[r, pl.ds(v * _L, _L)] = jnp.zeros((_L,), jnp.float32)
      return carry
    lax.fori_loop(0, k, zrow, 0)
    if packed_ea:
      def zmsg(r, carry):
        for v in range(dm // _L):
          z = jnp.zeros((_L,), jnp.float32)
          msgb0[r, pl.ds(dm + v * _L, _L)] = z
          msgb1[r, pl.ds(dm + v * _L, _L)] = z
        return carry
      lax.fori_loop(0, k, zmsg, 0)
    for t in range(rt // k):
      pltpu.sync_copy(xjb0, acc.at[pl.ds(s * rt + t * k, k)])
    plsc.subcore_barrier()

    pltpu.async_copy(x_hbm.at[srcb0], xjb0, gs0)
    wait_src(1)
    issue_gather_dst(1, 1)
    if not packed_ea:
      issue_ea(1, 1)
    wait_gather(0)
    wait_ea(0)
    issue_src(2, 0)
    compute(0)
    if packed_ea:
      issue_ea(1, 1)
    wait_dst(0)
    issue_scatter(0)

    def step(j, b):
      nb = 1 - b
      wait_scatter(nb)

      @pl.when(j < nch - 1)
      def _():
        wait_src(nb)
        issue_gather_dst(j + 1, nb)
        if not packed_ea:
          issue_ea(j + 1, nb)
      wait_gather(b)
      wait_ea(b)

      @pl.when(j < nch - 2)
      def _():
        issue_src(j + 2, b)
      compute(b)
      if packed_ea:
        @pl.when(j < nch - 1)
        def _():
          issue_ea(j + 1, nb)
      wait_dst(b)
      issue_scatter(b)

    head = (nch - 1) % 2
    if head:
      step(1, 1)
    start = 1 + head

    def pair(p, carry):
      for t in range(2):
        step(start + 2 * p + t, (start + t) % 2)
      return carry
    lax.fori_loop(0, (nch - 1 - head) // 2, pair, 0)
    wait_scatter((nch - 1) % 2)

    plsc.subcore_barrier()
    pltpu.sync_copy(acc.at[pl.ds(s * rt, rt)],
                    out_hbm.at[c, pl.ds(s * rt, rt)])

  return pl.kernel(
      body,
      out_type=jax.ShapeDtypeStruct((_NC, npad, d), jnp.float32),
      mesh=mesh,
      scratch_types=[
          pltpu.VMEM_SHARED((npad, d), jnp.float32),
          pltpu.VMEM((k,), jnp.int32),
          pltpu.VMEM((k,), jnp.int32),
          pltpu.VMEM((k,), jnp.int32),
          pltpu.VMEM((k,), jnp.int32),
          pltpu.VMEM((k, d), jnp.float32),
          pltpu.VMEM((k, d), jnp.float32),
          pltpu.VMEM((k, d), jnp.float32),
          pltpu.VMEM((k, d), jnp.float32),
          pltpu.VMEM((kk, d) if packed_ea else (8, _L), jnp.float32),
          pltpu.SemaphoreType.DMA,
          pltpu.SemaphoreType.DMA,
          pltpu.SemaphoreType.DMA,
          pltpu.SemaphoreType.DMA,
          pltpu.SemaphoreType.DMA,
          pltpu.SemaphoreType.DMA,
          pltpu.SemaphoreType.DMA,
          pltpu.SemaphoreType.DMA,
          pltpu.SemaphoreType.DMA,
          pltpu.SemaphoreType.DMA,
      ],
  )


def kernel(x, edge_index, edge_attr,
           lin1_W, lin1_b, m1_W1, m1_b1, m1_g, m1_be, m1_W2, m1_b2,
           lin2_W, lin2_b, m2_W1, m2_b1, m2_g, m2_be, m2_W2, m2_b2):
  n, d_in = x.shape
  e, h_dim = edge_attr.shape
  src = edge_index[0]
  dst = edge_index[1]

  pad = d_in - h_dim
  m2_W1p = jnp.concatenate([m2_W1, jnp.zeros((pad, m2_W1.shape[1]),
                                             jnp.float32)], 0)

  k2 = 80
  ea1, ea2p = _edge_lin(edge_attr, lin1_W, lin1_b, lin2_W, lin2_b, grp=k2)
  agg1 = _make_sc_conv(n, e, d_in, k=80, packed_ea=False)(x, src, dst, ea1)
  h = _mlp(agg1, x, m1_W1, m1_b1, m1_g, m1_be, m1_W2, m1_b2,
           final_relu=True, pad_to=d_in)
  agg2 = _make_sc_conv(n, e, d_in, k=k2, packed_ea=True)(h, src, dst, ea2p)
  return _mlp(agg2, h, m2_W1p, m2_b1, m2_g, m2_be, m2_W2, m2_b2,
              final_relu=False)


# speedup vs baseline: 1.0647x; 1.0003x over previous
import functools

import jax
import jax.numpy as jnp
from jax import lax
from jax.experimental import pallas as pl
from jax.experimental.pallas import tpu as pltpu
from jax.experimental.pallas import tpu_sc as plsc

_NC = 2
_NS = 16
_L = 16


def _pair_pack(o, blk, grp):
  w = o.shape[1]
  o3 = o.reshape(blk // grp, grp, w)
  packed = jnp.concatenate([o3[:, :grp // 2, :], o3[:, grp // 2:, :]], axis=2)
  return packed.reshape(blk // 2, 2 * w)


def _edge_lin_body(blk, grp, ea_ref, w1_ref, b1_ref, w2_ref, b2_ref,
                   o1_ref, o2_ref):
  a = ea_ref[...]
  o1 = jnp.dot(a, w1_ref[...], preferred_element_type=jnp.float32) + b1_ref[...]
  o1_ref[...] = o1
  o2 = jnp.dot(a, w2_ref[...], preferred_element_type=jnp.float32) + b2_ref[...]
  o2_ref[...] = _pair_pack(o2, blk, grp)


def _edge_lin(edge_attr, w1, b1, w2, b2, grp, blk=2560):
  e, h = edge_attr.shape
  d1 = w1.shape[1]
  d2 = w2.shape[1]
  assert blk % grp == 0 and grp % 2 == 0
  return pl.pallas_call(
      functools.partial(_edge_lin_body, blk, grp),
      grid=(e // blk,),
      in_specs=[
          pl.BlockSpec((blk, h), lambda i: (i, 0)),
          pl.BlockSpec((h, d1), lambda i: (0, 0)),
          pl.BlockSpec((1, d1), lambda i: (0, 0)),
          pl.BlockSpec((h, d2), lambda i: (0, 0)),
          pl.BlockSpec((1, d2), lambda i: (0, 0)),
      ],
      out_specs=[
          pl.BlockSpec((blk, d1), lambda i: (i, 0)),
          pl.BlockSpec((blk // 2, 2 * d2), lambda i: (i, 0)),
      ],
      out_shape=[
          jax.ShapeDtypeStruct((e, d1), jnp.float32),
          jax.ShapeDtypeStruct((e // 2, 2 * d2), jnp.float32),
      ],
  )(edge_attr, w1, b1.reshape(1, d1), w2, b2.reshape(1, d2))


def _mlp_body(d, final_relu, pad_to, agg_ref, x_ref, w1_ref, b1_ref, g_ref,
              be_ref, w2_ref, b2_ref, o_ref):
  x = x_ref[...]
  if x.shape[1] > d:
    x = x[:, :d]
  out = agg_ref[0] + agg_ref[1] + x
  h = jnp.dot(out, w1_ref[...], preferred_element_type=jnp.float32) + b1_ref[...]
  mu = jnp.mean(h, axis=0, keepdims=True)
  var = jnp.mean((h - mu) * (h - mu), axis=0, keepdims=True)
  h = (h - mu) / jnp.sqrt(var + 1e-5) * g_ref[...] + be_ref[...]
  h = jnp.maximum(h, 0.0)
  o = jnp.dot(h, w2_ref[...], preferred_element_type=jnp.float32) + b2_ref[...]
  if final_relu:
    o = jnp.maximum(o, 0.0)
  if pad_to > o.shape[1]:
    o = jnp.concatenate(
        [o, jnp.zeros((o.shape[0], pad_to - o.shape[1]), jnp.float32)], axis=1)
  o_ref[...] = o


def _mlp(agg, x, w1, b1, g, be, w2, b2, final_relu, pad_to=0):
  n = x.shape[0]
  dx = x.shape[1]
  d = w1.shape[0]
  dh = w1.shape[1]
  do = w2.shape[1]
  dout = max(do, pad_to)
  return pl.pallas_call(
      functools.partial(_mlp_body, d, final_relu, pad_to),
      grid=(1,),
      in_specs=[
          pl.BlockSpec((2, n, d), lambda i: (0, 0, 0)),
          pl.BlockSpec((n, dx), lambda i: (0, 0)),
          pl.BlockSpec((d, dh), lambda i: (0, 0)),
          pl.BlockSpec((1, dh), lambda i: (0, 0)),
          pl.BlockSpec((1, dh), lambda i: (0, 0)),
          pl.BlockSpec((1, dh), lambda i: (0, 0)),
          pl.BlockSpec((dh, do), lambda i: (0, 0)),
          pl.BlockSpec((1, do), lambda i: (0, 0)),
      ],
      out_specs=pl.BlockSpec((n, dout), lambda i: (0, 0)),
      out_shape=jax.ShapeDtypeStruct((n, dout), jnp.float32),
  )(agg, x, w1, b1.reshape(1, dh), g.reshape(1, dh), be.reshape(1, dh),
    w2, b2.reshape(1, do))


def _make_sc_conv(n, e, d, k, ea_mode):
  nw = _NC * _NS
  ew = e // nw
  nch = ew // k
  npad = -(-n // (k * _NS)) * (k * _NS)
  rt = npad // _NS
  grp = d // _L
  dm = d // 2
  kk = k // 2
  assert ew * nw == e and nch * k == ew and rt % k == 0
  assert nch >= 3 and k % 8 == 0 and k <= 128

  mesh = plsc.VectorSubcoreMesh(core_axis_name="c", subcore_axis_name="s",
                                num_cores=_NC, num_subcores=_NS)
  packed_ea = ea_mode == 'packed'
  single_ea = packed_ea
  ea_rows = kk if single_ea else k

  def body(x_hbm, src_hbm, dst_hbm, ea_hbm, out_hbm,
           acc, srcb0, srcb1, dstb0, dstb1, xjb0, xjb1, msgb0, msgb1,
           eab0, ps0, ps1, gs0, gs1, es0, es1, ds0, ds1, ss0, ss1):
    c = lax.axis_index("c")
    s = lax.axis_index("s")
    srcb = (srcb0, srcb1)
    dstb = (dstb0, dstb1)
    xjb = (xjb0, xjb1)
    msgb = (msgb0, msgb1)
    eab = (eab0, eab0) if single_ea else (msgb0, msgb1)
    es = (es0, es0) if single_ea else (es0, es1)
    ps = (ps0, ps1)
    gs = (gs0, gs1)
    ds = (ds0, ds1)
    ss = (ss0, ss1)
    base = (c * _NS + s) * ew
    base_ea = base // 2 if single_ea else base

    def issue_src(j, b):
      pltpu.async_copy(src_hbm.at[pl.ds(base + j * k, k)], srcb[b], ps[b])

    def wait_src(b):
      pltpu.make_async_copy(src_hbm.at[pl.ds(base, k)], srcb[b], ps[b]).wait()

    def issue_ea(j, b):
      off = pl.multiple_of(base_ea + j * ea_rows, 8)
      pltpu.async_copy(ea_hbm.at[pl.ds(off, ea_rows)], eab[b], es[b])

    def wait_ea(b):
      off = pl.multiple_of(base_ea, 8)
      pltpu.make_async_copy(ea_hbm.at[pl.ds(off, ea_rows)], eab[b],
                            es[b]).wait()

    def issue_dst(j, b):
      pltpu.async_copy(dst_hbm.at[pl.ds(base + j * k, k)], dstb[b], ds[b])

    def issue_gather_dst(j, b):
      pltpu.async_copy(x_hbm.at[srcb[b]], xjb[b], gs[b])
      issue_dst(j, b)

    def wait_gather(b):
      pltpu.make_async_copy(x_hbm.at[srcb[b]], xjb[b], gs[b]).wait()

    def wait_dst(b):
      pltpu.make_async_copy(dst_hbm.at[pl.ds(base, k)], dstb[b], ds[b]).wait()

    def compute(b):
      if packed_ea:
        @plsc.parallel_loop(0, kk)
        def _(r):
          for half in range(2):
            for v in range(dm // _L):
              so = pl.ds(v * _L, _L)
              eo = pl.ds(half * dm + v * _L, _L)
              msgb[b][half * kk + r, so] = jnp.maximum(
                  xjb[b][half * kk + r, so] + eab0[r, eo], 0.0)
      else:
        @plsc.parallel_loop(0, k)
        def _(r):
          for v in range(grp):
            sl = pl.ds(v * _L, _L)
            msgb[b][r, sl] = jnp.maximum(msgb[b][r, sl] + xjb[b][r, sl], 0.0)

    def issue_scatter(b):
      pltpu.async_copy(msgb[b], acc.at[dstb[b]], ss[b], add=True)

    def wait_scatter(b):
      pltpu.make_async_copy(msgb[b], acc.at[dstb[b]], ss[b]).wait()

    pltpu.sync_copy(src_hbm.at[pl.ds(base, k)], srcb0)
    issue_ea(0, 0)
    issue_dst(0, 0)
    issue_src(1, 1)

    def zrow(r, carry):
      for v in range(grp):
        xjb0[r, pl.ds(v * _L, _L)] = jnp.zeros((_L,), jnp.float32)
      return carry
    lax.fori_loop(0, k, zrow, 0)
    if packed_ea:
      def zmsg(r, carry):
        for v in range(dm // _L):
          z = jnp.zeros((_L,), jnp.float32)
          msgb0[r, pl.ds(dm + v * _L, _L)] = z
          msgb1[r, pl.ds(dm + v * _L, _L)] = z
        return carry
      lax.fori_loop(0, k, zmsg, 0)
    for t in range(rt // k):
      pltpu.sync_copy(xjb0, acc.at[pl.ds(s * rt + t * k, k)])
    plsc.subcore_barrier()

    pltpu.async_copy(x_hbm.at[srcb0], xjb0, gs0)
    wait_src(1)
    issue_gather_dst(1, 1)
    if not single_ea:
      issue_ea(1, 1)
    wait_gather(0)
    wait_ea(0)
    issue_src(2, 0)
    compute(0)
    if single_ea:
      issue_ea(1, 1)
    wait_dst(0)
    issue_scatter(0)

    def step(j, b):
      nb = 1 - b
      wait_scatter(nb)

      @pl.when(j < nch - 1)
      def _():
        wait_src(nb)
        issue_gather_dst(j + 1, nb)
        if not single_ea:
          issue_ea(j + 1, nb)
      wait_gather(b)
      wait_ea(b)

      @pl.when(j < nch - 2)
      def _():
        issue_src(j + 2, b)
      compute(b)
      if single_ea:
        @pl.when(j < nch - 1)
        def _():
          issue_ea(j + 1, nb)
      wait_dst(b)
      issue_scatter(b)

    head = (nch - 1) % 2
    if head:
      step(1, 1)
    start = 1 + head

    def pair(p, carry):
      for t in range(2):
        step(start + 2 * p + t, (start + t) % 2)
      return carry
    lax.fori_loop(0, (nch - 1 - head) // 2, pair, 0)
    wait_scatter((nch - 1) % 2)

    plsc.subcore_barrier()
    pltpu.sync_copy(acc.at[pl.ds(s * rt, rt)],
                    out_hbm.at[c, pl.ds(s * rt, rt)])

  return pl.kernel(
      body,
      out_type=jax.ShapeDtypeStruct((_NC, npad, d), jnp.float32),
      mesh=mesh,
      scratch_types=[
          pltpu.VMEM_SHARED((npad, d), jnp.float32),
          pltpu.VMEM((k,), jnp.int32),
          pltpu.VMEM((k,), jnp.int32),
          pltpu.VMEM((k,), jnp.int32),
          pltpu.VMEM((k,), jnp.int32),
          pltpu.VMEM((k, d), jnp.float32),
          pltpu.VMEM((k, d), jnp.float32),
          pltpu.VMEM((k, d), jnp.float32),
          pltpu.VMEM((k, d), jnp.float32),
          pltpu.VMEM((kk, d) if single_ea else (8, _L), jnp.float32),
          pltpu.SemaphoreType.DMA,
          pltpu.SemaphoreType.DMA,
          pltpu.SemaphoreType.DMA,
          pltpu.SemaphoreType.DMA,
          pltpu.SemaphoreType.DMA,
          pltpu.SemaphoreType.DMA,
          pltpu.SemaphoreType.DMA,
          pltpu.SemaphoreType.DMA,
          pltpu.SemaphoreType.DMA,
          pltpu.SemaphoreType.DMA,
      ],
  )


def kernel(x, edge_index, edge_attr,
           lin1_W, lin1_b, m1_W1, m1_b1, m1_g, m1_be, m1_W2, m1_b2,
           lin2_W, lin2_b, m2_W1, m2_b1, m2_g, m2_be, m2_W2, m2_b2):
  n, d_in = x.shape
  e, h_dim = edge_attr.shape
  src = edge_index[0]
  dst = edge_index[1]

  pad = d_in - h_dim
  m2_W1p = jnp.concatenate([m2_W1, jnp.zeros((pad, m2_W1.shape[1]),
                                             jnp.float32)], 0)

  k2 = 80
  ea1, ea2p = _edge_lin(edge_attr, lin1_W, lin1_b, lin2_W, lin2_b, grp=k2)
  agg1 = _make_sc_conv(n, e, d_in, k=80, ea_mode='f32')(x, src, dst, ea1)
  h = _mlp(agg1, x, m1_W1, m1_b1, m1_g, m1_be, m1_W2, m1_b2,
           final_relu=True, pad_to=d_in)
  agg2 = _make_sc_conv(n, e, d_in, k=k2, ea_mode='packed')(h, src, dst, ea2p)
  return _mlp(agg2, h, m2_W1p, m2_b1, m2_g, m2_be, m2_W2, m2_b2,
              final_relu=False)
